# Initial kernel scaffold; baseline (speedup 1.0000x reference)
import jax
import jax.numpy as jnp
from jax.experimental import pallas as pl


def kernel(x, edge_index, pin_feature, batch, macro_index, macro_pos, W1, b1, Wpin, Wm1a, bm1a, Wm1b, bm1b, Wm1c, bm1c, W2, b2, Wm2a, bm2a, Wm2b, bm2b, Wm2c, bm2c, W3, b3):
    raise NotImplementedError("write your pallas kernel here")


# speedup vs baseline: 4.3292x; 4.3292x over previous
import functools

import jax
import jax.numpy as jnp
from jax import lax
from jax.experimental import pallas as pl
from jax.experimental.pallas import tpu as pltpu
from jax.experimental.pallas import tpu_sc as plsc

N = 10000
E = 320000
NUM_HE = 5000
NUM_GRAPHS = 16
NHID = 128
HE_PAD = 5120
N_PAD = 10240
NW = 32
EPT = E // NW
CHUNK = 80
NCHUNK = EPT // CHUNK
NBLK = 10
BLK = N // NBLK

_HI = lax.Precision.HIGHEST


def _dot(a, b):
    return jnp.dot(a, b, preferred_element_type=jnp.float32, precision=_HI)


def _make_seg_sum(vals_rows, out_rows):
    rpt = out_rows // 16
    mesh = plsc.VectorSubcoreMesh(core_axis_name="c", subcore_axis_name="s")

    @functools.partial(
        pl.kernel,
        out_type=jax.ShapeDtypeStruct((2, out_rows, 128), jnp.float32),
        mesh=mesh,
        scratch_types=[
            pltpu.VMEM((NCHUNK, CHUNK), jnp.int32),
            pltpu.VMEM((NCHUNK, CHUNK), jnp.int32),
            pltpu.VMEM((CHUNK, 128), jnp.float32),
            pltpu.VMEM_SHARED((out_rows, 128), jnp.float32),
            pltpu.SemaphoreType.DMA,
        ],
    )
    def k(vals_hbm, ig_hbm, is_hbm, zero_hbm, out_hbm, ig_v, is_v, rows_v,
          acc, sem):
        cid = lax.axis_index("c")
        sid = lax.axis_index("s")
        wid = cid * 16 + sid
        base = sid * rpt
        pltpu.sync_copy(zero_hbm, acc.at[pl.ds(base, rpt)])
        pltpu.sync_copy(ig_hbm.at[wid], ig_v)
        pltpu.sync_copy(is_hbm.at[wid], is_v)
        plsc.subcore_barrier()

        def body(j, carry):
            pltpu.async_copy(vals_hbm.at[ig_v.at[j]], rows_v, sem).wait()
            pltpu.sync_copy(rows_v, acc.at[is_v.at[j]], add=True)
            return carry

        lax.fori_loop(0, NCHUNK, body, 0)
        plsc.subcore_barrier()
        pltpu.sync_copy(acc.at[pl.ds(base, rpt)],
                        out_hbm.at[cid, pl.ds(base, rpt)])

    return k


_sc_cache = {}


def _seg_to_he(*args):
    if "he" not in _sc_cache:
        _sc_cache["he"] = _make_seg_sum(N, HE_PAD)
    return _sc_cache["he"](*args)


def _seg_to_node(*args):
    if "node" not in _sc_cache:
        _sc_cache["node"] = _make_seg_sum(HE_PAD, N_PAD)
    return _sc_cache["node"](*args)


def _mm_body(x_ref, w_ref, o_ref):
    o_ref[...] = _dot(x_ref[...], w_ref[...])


def _mm(x, w):
    return pl.pallas_call(
        _mm_body,
        grid=(NBLK,),
        in_specs=[pl.BlockSpec((BLK, 128), lambda i: (i, 0)),
                  pl.BlockSpec((128, 128), lambda i: (0, 0))],
        out_specs=pl.BlockSpec((BLK, 128), lambda i: (i, 0)),
        out_shape=jax.ShapeDtypeStruct((N, 128), jnp.float32),
    )(x, w)


def _hefix_body(has_pin, pa_ref, pb_ref, aa_ref, ab_ref, wp_ref, o_ref):
    bv = aa_ref[:, 0] + ab_ref[:, 0]
    binv = jnp.where(bv > 0.0, 1.0 / bv, 0.0)
    he = pa_ref[...] + pb_ref[...]
    if has_pin:
        pm = aa_ref[:, 1:5] + ab_ref[:, 1:5]
        he = he + _dot(pm, wp_ref[...])
    o_ref[...] = he * binv[:, None]


def _hefix(pa, pb, aux, wpin, has_pin):
    hb = HE_PAD // 5
    return pl.pallas_call(
        functools.partial(_hefix_body, has_pin),
        grid=(5,),
        in_specs=[pl.BlockSpec((hb, 128), lambda i: (i, 0)),
                  pl.BlockSpec((hb, 128), lambda i: (i, 0)),
                  pl.BlockSpec((hb, 128), lambda i: (i, 0)),
                  pl.BlockSpec((hb, 128), lambda i: (i, 0)),
                  pl.BlockSpec((4, 128), lambda i: (0, 0))],
        out_specs=pl.BlockSpec((hb, 128), lambda i: (i, 0)),
        out_shape=jax.ShapeDtypeStruct((HE_PAD, 128), jnp.float32),
    )(pa, pb, aux[0], aux[1], wpin)


def _node_body(has_next, qa_ref, qb_ref, ca_ref, cb_ref, bt_ref, b_ref,
               wa_ref, ba_ref, wb_ref, bb_ref, wc_ref, bc_ref, wn_ref,
               rp_ref, *out_and_scratch):
    if has_next:
        y_ref, r_ref, mx_s, sm_s, cn_s = out_and_scratch
    else:
        r_ref, mx_s, sm_s, cn_s = out_and_scratch
    i = pl.program_id(0)
    deg = ca_ref[:, 0] + cb_ref[:, 0]
    dinv = jnp.where(deg > 0.0, 1.0 / deg, 0.0)
    xb = jax.nn.relu((qa_ref[...] + qb_ref[...]) * dinv[:, None] + b_ref[...])
    bv = bt_ref[0, 0, :]
    onehot = (bv[:, None] == lax.broadcasted_iota(jnp.int32, (BLK, 16), 1))
    onef = onehot.astype(jnp.float32)
    smg = lax.dot_general(onef, xb, (((0,), (0,)), ((), ())),
                          precision=_HI, preferred_element_type=jnp.float32)
    cng = jnp.broadcast_to(jnp.sum(onef, axis=0)[:, None], (16, 128))
    mxg = jnp.stack([
        jnp.max(jnp.where(onehot[:, g][:, None], xb, -jnp.inf), axis=0)
        for g in range(NUM_GRAPHS)], axis=0)

    @pl.when(i == 0)
    def _():
        mx_s[...] = mxg
        sm_s[...] = smg
        cn_s[...] = cng

    @pl.when(i > 0)
    def _():
        mx_s[...] = jnp.maximum(mx_s[...], mxg)
        sm_s[...] = sm_s[...] + smg
        cn_s[...] = cn_s[...] + cng

    @pl.when(i == NBLK - 1)
    def _():
        mean = sm_s[...] / jnp.clip(cn_s[...], 1.0)
        r_ref[...] = rp_ref[...] + jnp.concatenate([mx_s[...], mean], axis=1)

    if has_next:
        z = jax.nn.relu(_dot(xb, wa_ref[...]) + ba_ref[...])
        z = jax.nn.relu(_dot(z, wb_ref[...]) + bb_ref[...])
        z = jax.nn.relu(_dot(z, wc_ref[...]) + bc_ref[...])
        y_ref[...] = _dot(z, wn_ref[...])


def _node_stage(qparts, cnt_n, bt3, b, mlp_w, wnext, rprev, has_next):
    wa, ba, wb, bb, wc, bc = mlp_w
    row = lambda i: (i, 0)
    fix = lambda i: (0, 0)
    in_specs = [pl.BlockSpec((BLK, 128), row),
                pl.BlockSpec((BLK, 128), row),
                pl.BlockSpec((BLK, 128), row),
                pl.BlockSpec((BLK, 128), row),
                pl.BlockSpec((1, 1, BLK), lambda i: (i, 0, 0)),
                pl.BlockSpec((1, 128), fix),
                pl.BlockSpec((128, 128), fix),
                pl.BlockSpec((1, 128), fix),
                pl.BlockSpec((128, 128), fix),
                pl.BlockSpec((1, 128), fix),
                pl.BlockSpec((128, 128), fix),
                pl.BlockSpec((1, 128), fix),
                pl.BlockSpec((128, 128), fix),
                pl.BlockSpec((16, 256), fix)]
    scratch = [pltpu.VMEM((16, 128), jnp.float32)] * 3
    r_shape = jax.ShapeDtypeStruct((16, 256), jnp.float32)
    if has_next:
        out_specs = (pl.BlockSpec((BLK, 128), row), pl.BlockSpec((16, 256), fix))
        out_shape = (jax.ShapeDtypeStruct((N, 128), jnp.float32), r_shape)
    else:
        out_specs = pl.BlockSpec((16, 256), fix)
        out_shape = r_shape
    return pl.pallas_call(
        functools.partial(_node_body, has_next),
        grid=(NBLK,),
        in_specs=in_specs,
        out_specs=out_specs,
        out_shape=out_shape,
        scratch_shapes=scratch,
    )(qparts[0], qparts[1], cnt_n[0], cnt_n[1], bt3, b,
      wa, ba, wb, bb, wc, bc, wnext, rprev)


def kernel(x, edge_index, pin_feature, batch, macro_index, macro_pos,
           W1, b1, Wpin, Wm1a, bm1a, Wm1b, bm1b, Wm1c, bm1c,
           W2, b2, Wm2a, bm2a, Wm2b, bm2b, Wm2c, bm2c, W3, b3):
    src, dst = edge_index[0], edge_index[1]
    tmp = jnp.ones((macro_pos.shape[0], 1), dtype=macro_pos.dtype)
    mp = jnp.concatenate([macro_pos, tmp], axis=-1)
    pos = jnp.zeros((N, 3), dtype=x.dtype).at[macro_index].set(mp)
    x0 = jnp.concatenate([x, pos], axis=-1)

    ig3 = src.reshape(NW, NCHUNK, CHUNK)
    id3 = dst.reshape(NW, NCHUNK, CHUNK)
    eid3 = jnp.arange(E, dtype=jnp.int32).reshape(NW, NCHUNK, CHUNK)
    aux128 = jnp.concatenate(
        [jnp.ones((E, 1), jnp.float32), pin_feature,
         jnp.zeros((E, 123), jnp.float32)], axis=1)
    z_he = jnp.zeros((HE_PAD // 16, 128), jnp.float32)
    z_n = jnp.zeros((N_PAD // 16, 128), jnp.float32)
    bt3 = batch.reshape(NBLK, 1, BLK)

    cnt_he = _seg_to_he(aux128, eid3, id3, z_he)
    cnt_n = _seg_to_node(aux128, eid3, ig3, z_n)

    rzero = jnp.zeros((16, 256), jnp.float32)
    wzero = jnp.zeros((128, 128), jnp.float32)
    bzero = jnp.zeros((1, 128), jnp.float32)

    y = _mm(x0, W1)
    mlps = [(Wm1a, bm1a.reshape(1, 128), Wm1b, bm1b.reshape(1, 128),
             Wm1c, bm1c.reshape(1, 128)),
            (Wm2a, bm2a.reshape(1, 128), Wm2b, bm2b.reshape(1, 128),
             Wm2c, bm2c.reshape(1, 128)),
            (wzero, bzero, wzero, bzero, wzero, bzero)]
    wnexts = [W2, W3, wzero]
    biases = [b1.reshape(1, 128), b2.reshape(1, 128), b3.reshape(1, 128)]
    r = rzero
    for l in range(3):
        parts = _seg_to_he(y, ig3, id3, z_he)
        he = _hefix(parts[0], parts[1], cnt_he, Wpin, has_pin=(l == 0))
        qparts = _seg_to_node(he, id3, ig3, z_n)
        has_next = l < 2
        res = _node_stage(qparts, cnt_n, bt3, biases[l], mlps[l],
                          wnexts[l], r, has_next)
        if has_next:
            y, r = res
        else:
            r = res
    return r


# speedup vs baseline: 5.9068x; 1.3644x over previous
import functools

import jax
import jax.numpy as jnp
from jax import lax
from jax.experimental import pallas as pl
from jax.experimental.pallas import tpu as pltpu
from jax.experimental.pallas import tpu_sc as plsc

N = 10000
E = 320000
NUM_HE = 5000
NUM_GRAPHS = 16
NHID = 128
HE_PAD = 5120
N_PAD = 10240
NW = 32
EPT = E // NW
CHUNK = 80
NCHUNK = EPT // CHUNK
NBLK = 10
BLK = N // NBLK

_HI = lax.Precision.HIGHEST


def _dot(a, b):
    return jnp.dot(a, b, preferred_element_type=jnp.float32, precision=_HI)


def _make_seg_sum(out_rows, nbuf):
    rpt = out_rows // 16
    mesh = plsc.VectorSubcoreMesh(core_axis_name="c", subcore_axis_name="s")

    @functools.partial(
        pl.kernel,
        out_type=jax.ShapeDtypeStruct((2, out_rows, 128), jnp.float32),
        mesh=mesh,
        scratch_types=[
            pltpu.VMEM((NCHUNK, CHUNK), jnp.int32),
            pltpu.VMEM((NCHUNK, CHUNK), jnp.int32),
        ] + [pltpu.VMEM((CHUNK, 128), jnp.float32)] * nbuf
          + [pltpu.SemaphoreType.DMA] * (2 * nbuf)
          + [pltpu.VMEM_SHARED((out_rows, 128), jnp.float32)],
    )
    def k(vals_hbm, ig_hbm, is_hbm, zero_hbm, out_hbm, ig_v, is_v, *rest):
        rows = rest[:nbuf]
        gsem = rest[nbuf:2 * nbuf]
        ssem = rest[2 * nbuf:3 * nbuf]
        acc = rest[3 * nbuf]
        cid = lax.axis_index("c")
        sid = lax.axis_index("s")
        wid = cid * 16 + sid
        base = sid * rpt
        pltpu.sync_copy(zero_hbm, acc.at[pl.ds(base, rpt)])
        pltpu.sync_copy(ig_hbm.at[wid], ig_v)
        pltpu.sync_copy(is_hbm.at[wid], is_v)
        plsc.subcore_barrier()

        for b in range(nbuf):
            pltpu.async_copy(vals_hbm.at[ig_v.at[b]], rows[b], gsem[b])

        def group(g, carry):
            j0 = g * nbuf
            for b in range(nbuf):
                j = j0 + b
                pltpu.make_async_copy(vals_hbm.at[ig_v.at[j]], rows[b],
                                      gsem[b]).wait()
                pltpu.async_copy(rows[b], acc.at[is_v.at[j]], ssem[b],
                                 add=True)
                pltpu.make_async_copy(rows[b], acc.at[is_v.at[j]],
                                      ssem[b]).wait()
                pltpu.async_copy(vals_hbm.at[ig_v.at[j + nbuf]], rows[b],
                                 gsem[b])
            return carry

        lax.fori_loop(0, NCHUNK // nbuf - 1, group, 0)
        j0 = NCHUNK - nbuf
        for b in range(nbuf):
            j = j0 + b
            pltpu.make_async_copy(vals_hbm.at[ig_v.at[j]], rows[b],
                                  gsem[b]).wait()
            pltpu.async_copy(rows[b], acc.at[is_v.at[j]], ssem[b],
                             add=True)
            pltpu.make_async_copy(rows[b], acc.at[is_v.at[j]],
                                  ssem[b]).wait()
        plsc.subcore_barrier()
        pltpu.sync_copy(acc.at[pl.ds(base, rpt)],
                        out_hbm.at[cid, pl.ds(base, rpt)])

    return k


def _make_seg_sum_packed(out_rows, nbuf):
    rpt = out_rows // 16
    mesh = plsc.VectorSubcoreMesh(core_axis_name="c", subcore_axis_name="s")

    @functools.partial(
        pl.kernel,
        out_type=jax.ShapeDtypeStruct((2, out_rows, 128), jnp.float32),
        mesh=mesh,
        scratch_types=[
            pltpu.VMEM((NCHUNK, CHUNK), jnp.int32),
        ] + [pltpu.VMEM((CHUNK,), jnp.int32)] * (2 * nbuf)
          + [pltpu.VMEM((CHUNK, 128), jnp.float32)] * nbuf
          + [pltpu.SemaphoreType.DMA] * (2 * nbuf)
          + [pltpu.VMEM_SHARED((out_rows, 128), jnp.float32)],
    )
    def k(vals_hbm, pk_hbm, zero_hbm, out_hbm, pk_v, *rest):
        igc = rest[:nbuf]
        isc = rest[nbuf:2 * nbuf]
        rows = rest[2 * nbuf:3 * nbuf]
        gsem = rest[3 * nbuf:4 * nbuf]
        ssem = rest[4 * nbuf:5 * nbuf]
        acc = rest[5 * nbuf]
        cid = lax.axis_index("c")
        sid = lax.axis_index("s")
        wid = cid * 16 + sid
        base = sid * rpt
        pltpu.sync_copy(zero_hbm, acc.at[pl.ds(base, rpt)])
        pltpu.sync_copy(pk_hbm.at[wid], pk_v)
        plsc.subcore_barrier()

        def unpack(b, j):
            for t in range(CHUNK // 16):
                p = pk_v[j, pl.ds(t * 16, 16)]
                igc[b][pl.ds(t * 16, 16)] = lax.bitwise_and(p, 0xFFFF)
                isc[b][pl.ds(t * 16, 16)] = lax.shift_right_logical(p, 16)

        for b in range(nbuf):
            unpack(b, b)
            pltpu.async_copy(vals_hbm.at[igc[b]], rows[b], gsem[b])

        def group(g, carry):
            j0 = g * nbuf
            for b in range(nbuf):
                j = j0 + b
                pltpu.make_async_copy(vals_hbm.at[igc[b]], rows[b],
                                      gsem[b]).wait()
                pltpu.async_copy(rows[b], acc.at[isc[b]], ssem[b],
                                 add=True)
                pltpu.make_async_copy(rows[b], acc.at[isc[b]],
                                      ssem[b]).wait()
                unpack(b, j + nbuf)
                pltpu.async_copy(vals_hbm.at[igc[b]], rows[b], gsem[b])
            return carry

        lax.fori_loop(0, NCHUNK // nbuf - 1, group, 0)
        for b in range(nbuf):
            pltpu.make_async_copy(vals_hbm.at[igc[b]], rows[b],
                                  gsem[b]).wait()
            pltpu.async_copy(rows[b], acc.at[isc[b]], ssem[b], add=True)
            pltpu.make_async_copy(rows[b], acc.at[isc[b]], ssem[b]).wait()
        plsc.subcore_barrier()
        pltpu.sync_copy(acc.at[pl.ds(base, rpt)],
                        out_hbm.at[cid, pl.ds(base, rpt)])

    return k


def _make_count(out_rows, nbuf):
    rpt = out_rows // 16
    mesh = plsc.VectorSubcoreMesh(core_axis_name="c", subcore_axis_name="s")

    @functools.partial(
        pl.kernel,
        out_type=jax.ShapeDtypeStruct((2, out_rows, 128), jnp.float32),
        mesh=mesh,
        scratch_types=[
            pltpu.VMEM((NCHUNK, CHUNK), jnp.int32),
            pltpu.VMEM((CHUNK, 128), jnp.float32),
        ] + [pltpu.SemaphoreType.DMA] * nbuf
          + [pltpu.VMEM_SHARED((out_rows, 128), jnp.float32)],
    )
    def k(ones_hbm, is_hbm, zero_hbm, out_hbm, is_v, rows_v, *rest):
        ssem = rest[:nbuf]
        acc = rest[nbuf]
        cid = lax.axis_index("c")
        sid = lax.axis_index("s")
        wid = cid * 16 + sid
        base = sid * rpt
        pltpu.sync_copy(zero_hbm, acc.at[pl.ds(base, rpt)])
        pltpu.sync_copy(is_hbm.at[wid], is_v)
        pltpu.sync_copy(ones_hbm, rows_v)
        plsc.subcore_barrier()

        for b in range(nbuf):
            pltpu.async_copy(rows_v, acc.at[is_v.at[b]], ssem[b], add=True)

        def group(g, carry):
            j0 = g * nbuf
            for b in range(nbuf):
                j = j0 + b
                pltpu.make_async_copy(rows_v, acc.at[is_v.at[j]],
                                      ssem[b]).wait()
                pltpu.async_copy(rows_v, acc.at[is_v.at[j + nbuf]],
                                 ssem[b], add=True)
            return carry

        lax.fori_loop(0, NCHUNK // nbuf - 1, group, 0)
        j0 = NCHUNK - nbuf
        for b in range(nbuf):
            pltpu.make_async_copy(rows_v, acc.at[is_v.at[j0 + b]],
                                  ssem[b]).wait()
        plsc.subcore_barrier()
        pltpu.sync_copy(acc.at[pl.ds(base, rpt)],
                        out_hbm.at[cid, pl.ds(base, rpt)])

    return k


_sc_cache = {}


def _sc(name, maker, *args):
    if name not in _sc_cache:
        _sc_cache[name] = maker(*args)
    return _sc_cache[name]


def _seg_to_he(*a):
    return _sc("he", _make_seg_sum, HE_PAD, 5)(*a)


def _seg_to_node(*a):
    return _sc("node", _make_seg_sum, N_PAD, 1)(*a)


def _cnt_node(*a):
    return _sc("cntn", _make_count, N_PAD, 5)(*a)


def _mm_body(x_ref, w_ref, o_ref):
    o_ref[...] = _dot(x_ref[...], w_ref[...])


def _mm(x, w):
    return pl.pallas_call(
        _mm_body,
        grid=(NBLK,),
        in_specs=[pl.BlockSpec((BLK, 128), lambda i: (i, 0)),
                  pl.BlockSpec((128, 128), lambda i: (0, 0))],
        out_specs=pl.BlockSpec((BLK, 128), lambda i: (i, 0)),
        out_shape=jax.ShapeDtypeStruct((N, 128), jnp.float32),
    )(x, w)


def _hefix_body(has_pin, pa_ref, pb_ref, aa_ref, ab_ref, wp_ref, o_ref):
    bv = aa_ref[:, 0] + ab_ref[:, 0]
    binv = jnp.where(bv > 0.0, 1.0 / bv, 0.0)
    he = pa_ref[...] + pb_ref[...]
    if has_pin:
        pm = aa_ref[:, 1:5] + ab_ref[:, 1:5]
        he = he + _dot(pm, wp_ref[...])
    o_ref[...] = he * binv[:, None]


def _hefix(pa, pb, aux, wpin, has_pin):
    hb = HE_PAD // 5
    return pl.pallas_call(
        functools.partial(_hefix_body, has_pin),
        grid=(5,),
        in_specs=[pl.BlockSpec((hb, 128), lambda i: (i, 0)),
                  pl.BlockSpec((hb, 128), lambda i: (i, 0)),
                  pl.BlockSpec((hb, 128), lambda i: (i, 0)),
                  pl.BlockSpec((hb, 128), lambda i: (i, 0)),
                  pl.BlockSpec((4, 128), lambda i: (0, 0))],
        out_specs=pl.BlockSpec((hb, 128), lambda i: (i, 0)),
        out_shape=jax.ShapeDtypeStruct((HE_PAD, 128), jnp.float32),
    )(pa, pb, aux[0], aux[1], wpin)


def _node_body(has_next, qa_ref, qb_ref, ca_ref, cb_ref, bt_ref, b_ref,
               wa_ref, ba_ref, wb_ref, bb_ref, wc_ref, bc_ref, wn_ref,
               rp_ref, *out_and_scratch):
    if has_next:
        y_ref, r_ref, mx_s, sm_s, cn_s = out_and_scratch
    else:
        r_ref, mx_s, sm_s, cn_s = out_and_scratch
    i = pl.program_id(0)
    deg = ca_ref[:, 0] + cb_ref[:, 0]
    dinv = jnp.where(deg > 0.0, 1.0 / deg, 0.0)
    xb = jax.nn.relu((qa_ref[...] + qb_ref[...]) * dinv[:, None] + b_ref[...])
    bv = bt_ref[0, 0, :]
    onehot = (bv[:, None] == lax.broadcasted_iota(jnp.int32, (BLK, 16), 1))
    onef = onehot.astype(jnp.float32)
    smg = lax.dot_general(onef, xb, (((0,), (0,)), ((), ())),
                          precision=_HI, preferred_element_type=jnp.float32)
    cng = jnp.broadcast_to(jnp.sum(onef, axis=0)[:, None], (16, 128))
    mxg = jnp.stack([
        jnp.max(jnp.where(onehot[:, g][:, None], xb, -jnp.inf), axis=0)
        for g in range(NUM_GRAPHS)], axis=0)

    @pl.when(i == 0)
    def _():
        mx_s[...] = mxg
        sm_s[...] = smg
        cn_s[...] = cng

    @pl.when(i > 0)
    def _():
        mx_s[...] = jnp.maximum(mx_s[...], mxg)
        sm_s[...] = sm_s[...] + smg
        cn_s[...] = cn_s[...] + cng

    @pl.when(i == NBLK - 1)
    def _():
        mean = sm_s[...] / jnp.clip(cn_s[...], 1.0)
        r_ref[...] = rp_ref[...] + jnp.concatenate([mx_s[...], mean], axis=1)

    if has_next:
        z = jax.nn.relu(_dot(xb, wa_ref[...]) + ba_ref[...])
        z = jax.nn.relu(_dot(z, wb_ref[...]) + bb_ref[...])
        z = jax.nn.relu(_dot(z, wc_ref[...]) + bc_ref[...])
        y_ref[...] = _dot(z, wn_ref[...])


def _node_stage(qparts, cnt_n, bt3, b, mlp_w, wnext, rprev, has_next):
    wa, ba, wb, bb, wc, bc = mlp_w
    row = lambda i: (i, 0)
    fix = lambda i: (0, 0)
    in_specs = [pl.BlockSpec((BLK, 128), row),
                pl.BlockSpec((BLK, 128), row),
                pl.BlockSpec((BLK, 128), row),
                pl.BlockSpec((BLK, 128), row),
                pl.BlockSpec((1, 1, BLK), lambda i: (i, 0, 0)),
                pl.BlockSpec((1, 128), fix),
                pl.BlockSpec((128, 128), fix),
                pl.BlockSpec((1, 128), fix),
                pl.BlockSpec((128, 128), fix),
                pl.BlockSpec((1, 128), fix),
                pl.BlockSpec((128, 128), fix),
                pl.BlockSpec((1, 128), fix),
                pl.BlockSpec((128, 128), fix),
                pl.BlockSpec((16, 256), fix)]
    scratch = [pltpu.VMEM((16, 128), jnp.float32)] * 3
    r_shape = jax.ShapeDtypeStruct((16, 256), jnp.float32)
    if has_next:
        out_specs = (pl.BlockSpec((BLK, 128), row), pl.BlockSpec((16, 256), fix))
        out_shape = (jax.ShapeDtypeStruct((N, 128), jnp.float32), r_shape)
    else:
        out_specs = pl.BlockSpec((16, 256), fix)
        out_shape = r_shape
    return pl.pallas_call(
        functools.partial(_node_body, has_next),
        grid=(NBLK,),
        in_specs=in_specs,
        out_specs=out_specs,
        out_shape=out_shape,
        scratch_shapes=scratch,
    )(qparts[0], qparts[1], cnt_n[0], cnt_n[1], bt3, b,
      wa, ba, wb, bb, wc, bc, wnext, rprev)


def kernel(x, edge_index, pin_feature, batch, macro_index, macro_pos,
           W1, b1, Wpin, Wm1a, bm1a, Wm1b, bm1b, Wm1c, bm1c,
           W2, b2, Wm2a, bm2a, Wm2b, bm2b, Wm2c, bm2c, W3, b3):
    src, dst = edge_index[0], edge_index[1]
    tmp = jnp.ones((macro_pos.shape[0], 1), dtype=macro_pos.dtype)
    mp = jnp.concatenate([macro_pos, tmp], axis=-1)
    pos = jnp.zeros((N, 3), dtype=x.dtype).at[macro_index].set(mp)
    x0 = jnp.concatenate([x, pos], axis=-1)

    ig3 = src.reshape(NW, NCHUNK, CHUNK)
    id3 = dst.reshape(NW, NCHUNK, CHUNK)
    eid3 = jnp.arange(E, dtype=jnp.int32).reshape(NW, NCHUNK, CHUNK)
    pk3 = ((src << 16) | dst).reshape(NW, NCHUNK, CHUNK)
    aux128 = jnp.concatenate(
        [jnp.ones((E, 1), jnp.float32), pin_feature,
         jnp.zeros((E, 123), jnp.float32)], axis=1)
    ones128 = jnp.ones((CHUNK, 128), jnp.float32)
    z_he = jnp.zeros((HE_PAD // 16, 128), jnp.float32)
    z_n = jnp.zeros((N_PAD // 16, 128), jnp.float32)
    bt3 = batch.reshape(NBLK, 1, BLK)

    cnt_he = _seg_to_he(aux128, eid3, id3, z_he)
    z_d = lax.optimization_barrier((z_n, cnt_he))[0]
    cnt_n = _cnt_node(ones128, ig3, z_d)
    tok = cnt_n

    rzero = jnp.zeros((16, 256), jnp.float32)
    wzero = jnp.zeros((128, 128), jnp.float32)
    bzero = jnp.zeros((1, 128), jnp.float32)

    y = _mm(x0, W1)
    mlps = [(Wm1a, bm1a.reshape(1, 128), Wm1b, bm1b.reshape(1, 128),
             Wm1c, bm1c.reshape(1, 128)),
            (Wm2a, bm2a.reshape(1, 128), Wm2b, bm2b.reshape(1, 128),
             Wm2c, bm2c.reshape(1, 128)),
            (wzero, bzero, wzero, bzero, wzero, bzero)]
    wnexts = [W2, W3, wzero]
    biases = [b1.reshape(1, 128), b2.reshape(1, 128), b3.reshape(1, 128)]
    r = rzero
    for l in range(3):
        z_he_d = lax.optimization_barrier((z_he, tok))[0]
        parts = _seg_to_he(y, ig3, id3, z_he_d)
        he = _hefix(parts[0], parts[1], cnt_he, Wpin, has_pin=(l == 0))
        z_n_d = lax.optimization_barrier((z_n, parts))[0]
        qparts = _seg_to_node(he, id3, ig3, z_n_d)
        tok = qparts
        has_next = l < 2
        res = _node_stage(qparts, cnt_n, bt3, biases[l], mlps[l],
                          wnexts[l], r, has_next)
        if has_next:
            y, r = res
        else:
            r = res
    return r
